# baseline (speedup 1.0000x reference)
import jax
import jax.numpy as jnp
from jax import lax
from jax.experimental import pallas as pl
from jax.experimental.pallas import tpu as pltpu
from jax.experimental.pallas import tpu_sc as plsc

N = 10000
NP = 10240
E = 320000
D = 128
DH = 80

NC = 2
NS = 16
K = 112
CHUNKS = 180
EPT = CHUNKS * K
E_PAD = NS * EPT
NB = 3
ROWS_PER_SUB = NP // NS


def _mm_body(x_ref, w_ref, o0_ref, o1_ref):
    mm = jnp.dot(x_ref[...], w_ref[...], preferred_element_type=jnp.float32)
    bm = mm.shape[0]
    onehot = jnp.where(
        lax.broadcasted_iota(jnp.int32, (bm, DH - 48), 1) == 0, 1.0, 0.0
    ).astype(jnp.float32)
    o0_ref[...] = mm[:, :DH]
    o1_ref[...] = jnp.concatenate([mm[:, DH:], onehot], axis=1)


def _combine_body(p_ref, b_ref, o_ref):
    p = p_ref[...]
    s0 = p[0]
    s1 = p[1]
    deg = s1[:, 48:49]
    scale = 1.0 / jnp.maximum(deg, 1.0)
    o_ref[...] = jnp.concatenate([s0, s1[:, :48]], axis=1) * scale + b_ref[...]


def _sc_body(sup0_hbm, sup1_hbm, row_hbm, col_hbm, zeros_hbm, out_hbm,
             acc_sh, colv, rowv, rows_v, gsem, ssem):
    c = lax.axis_index("c")
    s = lax.axis_index("s")

    pltpu.async_copy(col_hbm.at[s], colv, gsem.at[0])
    pltpu.async_copy(row_hbm.at[s], rowv, gsem.at[1])
    sl = pl.ds(pl.multiple_of(s * ROWS_PER_SUB, 8), ROWS_PER_SUB)
    pltpu.sync_copy(zeros_hbm, acc_sh.at[sl])
    pltpu.make_async_copy(col_hbm.at[s], colv, gsem.at[0]).wait()
    pltpu.make_async_copy(row_hbm.at[s], rowv, gsem.at[1]).wait()
    plsc.subcore_barrier()

    def gather(i, buf):
        @pl.when(c == 0)
        def _():
            pltpu.async_copy(
                sup0_hbm.at[colv.at[i]], rows_v.at[buf], gsem.at[buf]
            )

        @pl.when(c == 1)
        def _():
            pltpu.async_copy(
                sup1_hbm.at[colv.at[i]], rows_v.at[buf], gsem.at[buf]
            )

    def gather_wait(i, buf):
        pltpu.make_async_copy(
            sup0_hbm.at[colv.at[i]], rows_v.at[buf], gsem.at[buf]
        ).wait()

    def scatter(i, buf):
        pltpu.async_copy(
            rows_v.at[buf], acc_sh.at[rowv.at[i]], ssem.at[buf], add=True
        )

    def scatter_wait(i, buf):
        pltpu.make_async_copy(
            rows_v.at[buf], acc_sh.at[rowv.at[i]], ssem.at[buf]
        ).wait()

    for j in range(NB - 1):
        gather(j, j)

    def body(i, carry):
        buf = lax.rem(i, NB)
        nxt = i + NB - 1

        @pl.when(nxt < CHUNKS)
        def _():
            nbuf = lax.rem(nxt, NB)

            @pl.when(nxt >= NB)
            def _():
                scatter_wait(nxt - NB, nbuf)

            gather(nxt, nbuf)

        gather_wait(i, buf)
        scatter(i, buf)
        return carry

    lax.fori_loop(0, CHUNKS, body, 0)
    for j in range(NB):
        scatter_wait(CHUNKS - NB + j, lax.rem(CHUNKS - NB + j, NB))
    plsc.subcore_barrier()

    pltpu.sync_copy(acc_sh.at[sl], out_hbm.at[c].at[sl])


_sc_scatter = pl.kernel(
    _sc_body,
    out_type=jax.ShapeDtypeStruct((NC, NP, DH), jnp.float32),
    mesh=plsc.VectorSubcoreMesh(core_axis_name="c", subcore_axis_name="s"),
    scratch_types=[
        pltpu.VMEM_SHARED((NP, DH), jnp.float32),
        pltpu.VMEM((CHUNKS, K), jnp.int32),
        pltpu.VMEM((CHUNKS, K), jnp.int32),
        pltpu.VMEM((NB, K, DH), jnp.float32),
        pltpu.SemaphoreType.DMA((NB,)),
        pltpu.SemaphoreType.DMA((NB,)),
    ],
    compiler_params=pltpu.CompilerParams(use_tc_tiling_on_sc=False),
)


def kernel(x, edge_index, W, b):
    npad = E_PAD - E
    row = jnp.concatenate(
        [edge_index[0], jnp.full((npad,), NP - 1, jnp.int32)]
    ).reshape(NS, CHUNKS, K)
    col = jnp.concatenate(
        [edge_index[1], jnp.zeros((npad,), jnp.int32)]
    ).reshape(NS, CHUNKS, K)

    bm = 2000
    sup0, sup1 = pl.pallas_call(
        _mm_body,
        grid=(N // bm,),
        in_specs=[
            pl.BlockSpec((bm, D), lambda i: (i, 0)),
            pl.BlockSpec((D, D), lambda i: (0, 0)),
        ],
        out_specs=[
            pl.BlockSpec((bm, DH), lambda i: (i, 0)),
            pl.BlockSpec((bm, DH), lambda i: (i, 0)),
        ],
        out_shape=[
            jax.ShapeDtypeStruct((N, DH), jnp.float32),
            jax.ShapeDtypeStruct((N, DH), jnp.float32),
        ],
    )(x, W)

    zeros = jnp.zeros((ROWS_PER_SUB, DH), jnp.float32)
    partials = _sc_scatter(sup0, sup1, row, col, zeros)

    b2 = b.reshape(1, D)
    out = pl.pallas_call(
        _combine_body,
        grid=(N // bm,),
        in_specs=[
            pl.BlockSpec((NC, bm, DH), lambda i: (0, i, 0)),
            pl.BlockSpec((1, D), lambda i: (0, 0)),
        ],
        out_specs=pl.BlockSpec((bm, D), lambda i: (i, 0)),
        out_shape=jax.ShapeDtypeStruct((N, D), jnp.float32),
    )(partials, b2)
    return out


# speedup vs baseline: 1.2975x; 1.2975x over previous
import jax
import jax.numpy as jnp
from jax import lax
from jax.experimental import pallas as pl
from jax.experimental.pallas import tpu as pltpu
from jax.experimental.pallas import tpu_sc as plsc

N = 10000
NP = 10240
E = 320000
D = 128
DE = 144

NC = 2
NS = 16
NW = NC * NS
EPT = E // NW
K = 40
CHUNKS = EPT // K
NB = 3
ROWS_PER_SUB = NP // NS


def _mm_body(x_ref, w_ref, o_ref):
    mm = jnp.dot(x_ref[...], w_ref[...], preferred_element_type=jnp.float32)
    bm = mm.shape[0]
    onehot = jnp.where(
        lax.broadcasted_iota(jnp.int32, (bm, DE - D), 1) == 0, 1.0, 0.0
    ).astype(jnp.float32)
    o_ref[...] = jnp.concatenate([mm, onehot], axis=1)


def _combine_body(p_ref, b_ref, o_ref):
    p = p_ref[...]
    s = p[0] + p[1]
    deg = s[:, D:D + 1]
    scale = 1.0 / jnp.maximum(deg, 1.0)
    o_ref[...] = s[:, :D] * scale + b_ref[...]


def _sc_body(sup_hbm, row_hbm, col_hbm, zeros_hbm, out_hbm,
             acc_sh, colv, rowv, rows_v, gsem, ssem):
    c = lax.axis_index("c")
    s = lax.axis_index("s")
    wid = s * NC + c

    pltpu.async_copy(col_hbm.at[wid], colv, gsem.at[0])
    pltpu.async_copy(row_hbm.at[wid], rowv, gsem.at[1])
    sl = pl.ds(pl.multiple_of(s * ROWS_PER_SUB, 8), ROWS_PER_SUB)
    pltpu.sync_copy(zeros_hbm, acc_sh.at[sl])
    pltpu.make_async_copy(col_hbm.at[wid], colv, gsem.at[0]).wait()
    pltpu.make_async_copy(row_hbm.at[wid], rowv, gsem.at[1]).wait()
    plsc.subcore_barrier()

    def gather(i, buf):
        pltpu.async_copy(sup_hbm.at[colv.at[i]], rows_v.at[buf], gsem.at[buf])

    def gather_wait(i, buf):
        pltpu.make_async_copy(
            sup_hbm.at[colv.at[i]], rows_v.at[buf], gsem.at[buf]
        ).wait()

    def scatter(i, buf):
        pltpu.async_copy(
            rows_v.at[buf], acc_sh.at[rowv.at[i]], ssem.at[buf], add=True
        )

    def scatter_wait(i, buf):
        pltpu.make_async_copy(
            rows_v.at[buf], acc_sh.at[rowv.at[i]], ssem.at[buf]
        ).wait()

    for j in range(NB - 1):
        gather(j, j)

    def body(i, carry):
        buf = lax.rem(i, NB)
        nxt = i + NB - 1

        @pl.when(nxt < CHUNKS)
        def _():
            nbuf = lax.rem(nxt, NB)

            @pl.when(nxt >= NB)
            def _():
                scatter_wait(nxt - NB, nbuf)

            gather(nxt, nbuf)

        gather_wait(i, buf)
        scatter(i, buf)
        return carry

    lax.fori_loop(0, CHUNKS, body, 0)
    for j in range(NB):
        scatter_wait(CHUNKS - NB + j, lax.rem(CHUNKS - NB + j, NB))
    plsc.subcore_barrier()

    pltpu.sync_copy(acc_sh.at[sl], out_hbm.at[c].at[sl])


_sc_scatter = pl.kernel(
    _sc_body,
    out_type=jax.ShapeDtypeStruct((NC, NP, DE), jnp.float32),
    mesh=plsc.VectorSubcoreMesh(core_axis_name="c", subcore_axis_name="s"),
    scratch_types=[
        pltpu.VMEM_SHARED((NP, DE), jnp.float32),
        pltpu.VMEM((CHUNKS, K), jnp.int32),
        pltpu.VMEM((CHUNKS, K), jnp.int32),
        pltpu.VMEM((NB, K, DE), jnp.float32),
        pltpu.SemaphoreType.DMA((NB,)),
        pltpu.SemaphoreType.DMA((NB,)),
    ],
    compiler_params=pltpu.CompilerParams(use_tc_tiling_on_sc=False),
)


def kernel(x, edge_index, W, b):
    row = edge_index[0].reshape(NW, CHUNKS, K)
    col = edge_index[1].reshape(NW, CHUNKS, K)

    bm = 2000
    support_ext = pl.pallas_call(
        _mm_body,
        grid=(N // bm,),
        in_specs=[
            pl.BlockSpec((bm, D), lambda i: (i, 0)),
            pl.BlockSpec((D, D), lambda i: (0, 0)),
        ],
        out_specs=pl.BlockSpec((bm, DE), lambda i: (i, 0)),
        out_shape=jax.ShapeDtypeStruct((N, DE), jnp.float32),
    )(x, W)

    zeros = jnp.zeros((ROWS_PER_SUB, DE), jnp.float32)
    partials = _sc_scatter(support_ext, row, col, zeros)

    b2 = b.reshape(1, D)
    out = pl.pallas_call(
        _combine_body,
        grid=(N // bm,),
        in_specs=[
            pl.BlockSpec((NC, bm, DE), lambda i: (0, i, 0)),
            pl.BlockSpec((1, D), lambda i: (0, 0)),
        ],
        out_specs=pl.BlockSpec((bm, D), lambda i: (i, 0)),
        out_shape=jax.ShapeDtypeStruct((N, D), jnp.float32),
    )(partials, b2)
    return out
